# baseline (speedup 1.0000x reference)
import jax
import jax.numpy as jnp
from jax import lax
from jax.experimental import pallas as pl
from jax.experimental.pallas import tpu as pltpu
from jax.experimental.pallas import tpu_sc as plsc

_B, _L, _D = 4, 2048, 768
_NPOS = 30
_NW = 32
_ROWS = _B * _L
_RPW = _ROWS // _NW
_CHUNK = 64
_NCHUNK = _RPW // _CHUNK


def _sc_body(x_hbm, idx_hbm, tab_hbm, out_hbm,
             idx_s, idx_v, tab_v, buf0, buf1, sin0, sin1, sout0, sout1):
    wid = lax.axis_index("s") * 2 + lax.axis_index("c")
    base = wid * _RPW
    pltpu.sync_copy(idx_hbm.at[pl.ds(base, _RPW)], idx_v)
    for i in range(_NPOS + 2):
        pltpu.sync_copy(tab_hbm.at[i, :], tab_v.at[pl.ds(i * _D, _D)])

    bufs = (buf0, buf1)
    sins = (sin0, sin1)
    souts = (sout0, sout1)

    def start_load(c):
        b = c & 1
        r0 = base + c * _CHUNK
        return pltpu.async_copy(
            x_hbm.at[pl.ds(r0, _CHUNK), :], bufs[b], sins[b])

    def start_store(c):
        b = c & 1
        r0 = base + c * _CHUNK
        return pltpu.async_copy(
            bufs[b], out_hbm.at[pl.ds(r0, _CHUNK), :], souts[b])

    loads = {0: start_load(0)}
    stores = {}
    for c in range(_NCHUNK):
        b = c & 1
        loads.pop(c).wait()
        buf = bufs[b]

        def grp_body(g, carry, _c=c, _buf=buf):
            tvec = idx_v[pl.ds(_c * _CHUNK + g * 16, 16)]
            for k in range(0, 16, 2):
                s0 = tvec[k] * _D
                s1 = tvec[k + 1] * _D
                r0 = g * 16 + k
                r1 = g * 16 + k + 1

                @plsc.parallel_loop(0, _D, 16, unroll=8)
                def jbody(j, _s0=s0, _s1=s1, _r0=r0, _r1=r1, _b=_buf):
                    v0 = tab_v[pl.ds(_s0 + j, 16)]
                    plsc.addupdate(_b.at[_r0, pl.ds(j, 16)], v0)
                    v1 = tab_v[pl.ds(_s1 + j, 16)]
                    plsc.addupdate(_b.at[_r1, pl.ds(j, 16)], v1)

            return carry

        lax.fori_loop(0, _CHUNK // 16, grp_body, 0)

        stores[c] = start_store(c)
        if c + 1 < _NCHUNK:
            if c - 1 >= 0:
                stores.pop(c - 1).wait()
            loads[c + 1] = start_load(c + 1)
    stores.pop(_NCHUNK - 1).wait()


def kernel(inputs, times, pos_table):
    x = inputs.reshape(_ROWS, _D)
    idx = times.astype(jnp.int32)
    col = lax.broadcasted_iota(jnp.int32, (_B, _L), 1)
    idx = jnp.where(col == 0, _NPOS, idx).reshape(_ROWS)
    tab = jnp.concatenate(
        [pos_table.astype(jnp.float32), jnp.zeros((2, _D), jnp.float32)],
        axis=0,
    )

    mesh = plsc.VectorSubcoreMesh(core_axis_name="c", subcore_axis_name="s")
    f = pl.kernel(
        _sc_body,
        out_type=jax.ShapeDtypeStruct((_ROWS, _D), jnp.float32),
        mesh=mesh,
        compiler_params=pltpu.CompilerParams(
            use_tc_tiling_on_sc=True, needs_layout_passes=False
        ),
        scratch_types=[
            pltpu.SMEM((_RPW,), jnp.int32),
            pltpu.VMEM((_RPW,), jnp.int32),
            pltpu.VMEM(((_NPOS + 2) * _D,), jnp.float32),
            pltpu.VMEM((_CHUNK, _D), jnp.float32),
            pltpu.VMEM((_CHUNK, _D), jnp.float32),
            pltpu.SemaphoreType.DMA,
            pltpu.SemaphoreType.DMA,
            pltpu.SemaphoreType.DMA,
            pltpu.SemaphoreType.DMA,
        ],
    )
    out = f(x, idx, tab)
    return out.reshape(_B, _L, _D)


# speedup vs baseline: 1.0503x; 1.0503x over previous
import jax
import jax.numpy as jnp
from jax import lax
from jax.experimental import pallas as pl
from jax.experimental.pallas import tpu as pltpu
from jax.experimental.pallas import tpu_sc as plsc

_B, _L, _D = 4, 2048, 768
_NPOS = 30
_NW = 32
_ROWS = _B * _L
_RPW = _ROWS // _NW
_CHUNK = 64
_NCHUNK = _RPW // _CHUNK
_CT = _D // 128


def _sc_body(x_hbm, idx_hbm, tab_hbm, out_hbm,
             idx_v, tab_v, buf0, buf1, sin0, sin1, sout0, sout1):
    wid = lax.axis_index("s") * 2 + lax.axis_index("c")
    base = wid * _RPW
    pltpu.sync_copy(idx_hbm.at[pl.ds(base, _RPW)], idx_v)
    bufs = (buf0, buf1)
    sins = (sin0, sin1)
    souts = (sout0, sout1)
    iota = lax.iota(jnp.int32, 16)
    for i in range(_NPOS + 2):
        pltpu.sync_copy(tab_hbm.at[i, :], tab_v.at[pl.ds(i * _D, _D)])

    def start_load(c):
        b = c & 1
        r0 = base + c * _CHUNK
        return pltpu.async_copy(
            x_hbm.at[pl.ds(r0, _CHUNK), :], bufs[b], sins[b])

    def start_store(c):
        b = c & 1
        r0 = base + c * _CHUNK
        return pltpu.async_copy(
            bufs[b], out_hbm.at[pl.ds(r0, _CHUNK), :], souts[b])

    loads = {0: start_load(0)}
    stores = {}
    for c in range(_NCHUNK):
        b = c & 1
        loads.pop(c).wait()
        buf = bufs[b]

        def row_body(rp, carry, _c=c, _buf=buf):
            rr = [rp * 4 + k for k in range(4)]
            ts = [plsc.load_gather(
                idx_v, [jnp.broadcast_to(_c * _CHUNK + r, (16,))])
                for r in rr]
            aa = [t * _D + iota for t in ts]

            @plsc.parallel_loop(0, _D, 16, unroll=4)
            def jbody(j, _aa=aa, _rr=rr, _b=_buf):
                jv = jnp.broadcast_to(j, (16,)).astype(jnp.int32)
                for k in range(4):
                    v = plsc.load_gather(tab_v, [_aa[k] + jv])
                    plsc.addupdate(_b.at[_rr[k], pl.ds(j, 16)], v)

            return carry

        lax.fori_loop(0, _CHUNK // 4, row_body, 0)

        stores[c] = start_store(c)
        if c + 1 < _NCHUNK:
            if c - 1 >= 0:
                stores.pop(c - 1).wait()
            loads[c + 1] = start_load(c + 1)
    stores.pop(_NCHUNK - 1).wait()


def kernel(inputs, times, pos_table):
    x = inputs.reshape(_ROWS, _D)
    idx = times.astype(jnp.int32)
    col = lax.broadcasted_iota(jnp.int32, (_B, _L), 1)
    idx = jnp.where(col == 0, _NPOS, idx).reshape(_ROWS)
    tab = jnp.concatenate(
        [pos_table.astype(jnp.float32), jnp.zeros((2, _D), jnp.float32)],
        axis=0,
    )

    mesh = plsc.VectorSubcoreMesh(core_axis_name="c", subcore_axis_name="s")
    f = pl.kernel(
        _sc_body,
        out_type=jax.ShapeDtypeStruct((_ROWS, _D), jnp.float32),
        mesh=mesh,
        compiler_params=pltpu.CompilerParams(
            use_tc_tiling_on_sc=True, needs_layout_passes=False
        ),
        scratch_types=[
            pltpu.VMEM((_RPW,), jnp.int32),
            pltpu.VMEM(((_NPOS + 2) * _D,), jnp.float32),
            pltpu.VMEM((_CHUNK, _D), jnp.float32),
            pltpu.VMEM((_CHUNK, _D), jnp.float32),
            pltpu.SemaphoreType.DMA,
            pltpu.SemaphoreType.DMA,
            pltpu.SemaphoreType.DMA,
            pltpu.SemaphoreType.DMA,
        ],
    )
    out = f(x, idx, tab)
    return out.reshape(_B, _L, _D)


# speedup vs baseline: 1.0692x; 1.0180x over previous
import jax
import jax.numpy as jnp
from jax import lax
from jax.experimental import pallas as pl
from jax.experimental.pallas import tpu as pltpu
from jax.experimental.pallas import tpu_sc as plsc

_B, _L, _D = 4, 2048, 768
_NPOS = 30
_NW = 32
_ROWS = _B * _L
_RPW = _ROWS // _NW
_CHUNK = 64
_NCHUNK = _RPW // _CHUNK
_CT = _D // 128


def _sc_body(x_hbm, idx_hbm, tab_hbm, out_hbm,
             idx_v, tab_v, buf0, buf1, sin0, sin1, sout0, sout1):
    wid = lax.axis_index("s") * 2 + lax.axis_index("c")
    base = wid * _RPW
    pltpu.sync_copy(idx_hbm.at[pl.ds(base, _RPW)], idx_v)
    bufs = (buf0, buf1)
    sins = (sin0, sin1)
    souts = (sout0, sout1)
    iota = lax.iota(jnp.int32, 16)
    for i in range(_NPOS):
        pltpu.sync_copy(tab_hbm.at[i, :], tab_v.at[pl.ds(i * _D, _D)])
    zv = jnp.zeros((16,), jnp.float32)
    for k in range(_NPOS * _D, (_NPOS + 2) * _D, 16):
        tab_v[pl.ds(k, 16)] = zv

    def start_load(c):
        b = c & 1
        r0 = base + c * _CHUNK
        return pltpu.async_copy(
            x_hbm.at[pl.ds(r0, _CHUNK), :], bufs[b], sins[b])

    def start_store(c):
        b = c & 1
        r0 = base + c * _CHUNK
        return pltpu.async_copy(
            bufs[b], out_hbm.at[pl.ds(r0, _CHUNK), :], souts[b])

    loads = {0: start_load(0)}
    stores = {}
    for c in range(_NCHUNK):
        b = c & 1
        loads.pop(c).wait()
        buf = bufs[b]

        def row_body(rp, carry, _c=c, _buf=buf):
            rr = [rp * 4 + k for k in range(4)]
            ts = [plsc.load_gather(
                idx_v, [jnp.broadcast_to(_c * _CHUNK + r, (16,))])
                for r in rr]
            pad = jnp.full((16,), _NPOS, jnp.int32)
            ts = [
                jnp.where(
                    jnp.broadcast_to(
                        ((base + _c * _CHUNK + r) & (_L - 1)) == 0, (16,)),
                    pad, t)
                for r, t in zip(rr, ts)
            ]
            aa = [t * _D + iota for t in ts]

            @plsc.parallel_loop(0, _D, 16, unroll=4)
            def jbody(j, _aa=aa, _rr=rr, _b=_buf):
                jv = jnp.broadcast_to(j, (16,)).astype(jnp.int32)
                for k in range(4):
                    v = plsc.load_gather(tab_v, [_aa[k] + jv])
                    plsc.addupdate(_b.at[_rr[k], pl.ds(j, 16)], v)

            return carry

        lax.fori_loop(0, _CHUNK // 4, row_body, 0)

        stores[c] = start_store(c)
        if c + 1 < _NCHUNK:
            if c - 1 >= 0:
                stores.pop(c - 1).wait()
            loads[c + 1] = start_load(c + 1)
    stores.pop(_NCHUNK - 1).wait()


def kernel(inputs, times, pos_table):
    x = inputs.reshape(_ROWS, _D)
    idx = times.astype(jnp.int32).reshape(_ROWS)
    tab = pos_table.astype(jnp.float32)

    mesh = plsc.VectorSubcoreMesh(core_axis_name="c", subcore_axis_name="s")
    f = pl.kernel(
        _sc_body,
        out_type=jax.ShapeDtypeStruct((_ROWS, _D), jnp.float32),
        mesh=mesh,
        compiler_params=pltpu.CompilerParams(
            use_tc_tiling_on_sc=True, needs_layout_passes=False
        ),
        scratch_types=[
            pltpu.VMEM((_RPW,), jnp.int32),
            pltpu.VMEM(((_NPOS + 2) * _D,), jnp.float32),
            pltpu.VMEM((_CHUNK, _D), jnp.float32),
            pltpu.VMEM((_CHUNK, _D), jnp.float32),
            pltpu.SemaphoreType.DMA,
            pltpu.SemaphoreType.DMA,
            pltpu.SemaphoreType.DMA,
            pltpu.SemaphoreType.DMA,
        ],
    )
    out = f(x, idx, tab)
    return out.reshape(_B, _L, _D)


# speedup vs baseline: 1.3112x; 1.2263x over previous
import jax
import jax.numpy as jnp
from jax import lax
from jax.experimental import pallas as pl
from jax.experimental.pallas import tpu as pltpu
from jax.experimental.pallas import tpu_sc as plsc

_B, _L, _D = 4, 2048, 768
_NPOS = 30
_NW = 32
_ROWS = _B * _L
_RPW = _ROWS // _NW
_CHUNK = 64
_NCHUNK = _RPW // _CHUNK
_CT = _D // 128


def _sc_body(x_hbm, idx_hbm, tab_hbm, out_hbm,
             idx_v, tab_v, buf0, buf1, sin0, sin1, sout0, sout1, stab):
    wid = lax.axis_index("s") * 2 + lax.axis_index("c")
    base = wid * _RPW
    bufs = (buf0, buf1)
    sins = (sin0, sin1)
    souts = (sout0, sout1)
    iota = lax.iota(jnp.int32, 16)

    def start_load(c):
        b = c & 1
        r0 = base + c * _CHUNK
        return pltpu.async_copy(
            x_hbm.at[pl.ds(r0, _CHUNK), :], bufs[b], sins[b])

    def start_store(c):
        b = c & 1
        r0 = base + c * _CHUNK
        return pltpu.async_copy(
            bufs[b], out_hbm.at[pl.ds(r0, _CHUNK), :], souts[b])

    loads = {0: start_load(0)}
    tcopies = [
        pltpu.async_copy(tab_hbm.at[i, :], tab_v.at[pl.ds(i * _D, _D)], stab)
        for i in range(_NPOS)
    ]
    pltpu.sync_copy(idx_hbm.at[pl.ds(base, _RPW)], idx_v)
    zv = jnp.zeros((16,), jnp.float32)
    for k in range(_NPOS * _D, (_NPOS + 2) * _D, 16):
        tab_v[pl.ds(k, 16)] = zv
    for t in tcopies:
        t.wait()
    stores = {}
    for c in range(_NCHUNK):
        b = c & 1
        loads.pop(c).wait()
        buf = bufs[b]

        def row_body(rp, carry, _c=c, _buf=buf):
            rr = [rp * 4 + k for k in range(4)]
            ts = [plsc.load_gather(
                idx_v, [jnp.broadcast_to(_c * _CHUNK + r, (16,))])
                for r in rr]
            pad = jnp.full((16,), _NPOS, jnp.int32)
            ts = [
                jnp.where(
                    jnp.broadcast_to(
                        ((base + _c * _CHUNK + r) & (_L - 1)) == 0, (16,)),
                    pad, t)
                for r, t in zip(rr, ts)
            ]
            aa = [t * _D + iota for t in ts]

            @plsc.parallel_loop(0, _D, 16, unroll=4)
            def jbody(j, _aa=aa, _rr=rr, _b=_buf):
                jv = jnp.broadcast_to(j, (16,)).astype(jnp.int32)
                for k in range(4):
                    v = plsc.load_gather(tab_v, [_aa[k] + jv])
                    plsc.addupdate(_b.at[_rr[k], pl.ds(j, 16)], v)

            return carry

        lax.fori_loop(0, _CHUNK // 4, row_body, 0)

        stores[c] = start_store(c)
        if c + 1 < _NCHUNK:
            if c - 1 >= 0:
                stores.pop(c - 1).wait()
            loads[c + 1] = start_load(c + 1)
    stores.pop(_NCHUNK - 1).wait()


def kernel(inputs, times, pos_table):
    x = inputs.reshape(_ROWS, _D)
    idx = times.astype(jnp.int32).reshape(_ROWS)
    tab = pos_table.astype(jnp.float32)

    mesh = plsc.VectorSubcoreMesh(core_axis_name="c", subcore_axis_name="s")
    f = pl.kernel(
        _sc_body,
        out_type=jax.ShapeDtypeStruct((_ROWS, _D), jnp.float32),
        mesh=mesh,
        compiler_params=pltpu.CompilerParams(
            use_tc_tiling_on_sc=True, needs_layout_passes=False
        ),
        scratch_types=[
            pltpu.VMEM((_RPW,), jnp.int32),
            pltpu.VMEM(((_NPOS + 2) * _D,), jnp.float32),
            pltpu.VMEM((_CHUNK, _D), jnp.float32),
            pltpu.VMEM((_CHUNK, _D), jnp.float32),
            pltpu.SemaphoreType.DMA,
            pltpu.SemaphoreType.DMA,
            pltpu.SemaphoreType.DMA,
            pltpu.SemaphoreType.DMA,
            pltpu.SemaphoreType.DMA,
        ],
    )
    out = f(x, idx, tab)
    return out.reshape(_B, _L, _D)
